# baseline (speedup 1.0000x reference)
import functools

import jax
import jax.numpy as jnp
from jax.experimental import pallas as pl
from jax.experimental.pallas import tpu as pltpu

N = 10000
D = 128
C = 16
K = 5000
R = 10
M = N // R


def _body(x_ref, w_ref, b_ref, wp_ref, xf_ref, predt_ref, pts_ref, zs_ref):
    j = pl.program_id(0)
    x = x_ref[:, :]
    w = w_ref[:, :]
    bt = b_ref[:, :]
    wp = wp_ref[:, :]

    pt = jax.lax.dot_general(
        w, x, (((1,), (1,)), ((), ())), preferred_element_type=jnp.float32
    ) + bt
    pts_ref[j] = pt

    zj = jax.lax.dot_general(
        wp, pt, (((1,), (0,)), ((), ())),
        preferred_element_type=jnp.float32)
    zs_ref[pl.ds(j, 1), :] = zj

    @pl.when(j == R - 1)
    def _finale():
        zd = zs_ref[:, :]

        u = jax.lax.bitcast_convert_type(zd, jnp.uint32)
        sign = u >> jnp.uint32(31)
        flip = jnp.where(sign == jnp.uint32(1),
                         jnp.uint32(0xFFFFFFFF), jnp.uint32(0x80000000))
        key = u ^ flip

        def _cnt_ge(t):
            return jnp.sum((key >= t).astype(jnp.int32))

        kth = jnp.uint32(0)
        for sh in range(28, -1, -4):
            cnts = [_cnt_ge(kth | jnp.uint32(d << sh)) for d in range(1, 16)]
            digit = sum((c >= K).astype(jnp.uint32) for c in cnts)
            kth = kth | (digit << jnp.uint32(sh))

        above = key > kth
        m = jnp.sum(above.astype(jnp.int32))
        need = K - m

        tie = key == kth
        idx = (jax.lax.broadcasted_iota(jnp.int32, (R, M), 0) * M
               + jax.lax.broadcasted_iota(jnp.int32, (R, M), 1))

        def _cnt_lt(v):
            return jnp.sum((tie & (idx < v)).astype(jnp.int32))

        jstar = jnp.int32(0)
        for sh in range(12, -1, -4):
            cnts = [_cnt_lt(jstar | jnp.int32(d << sh)) for d in range(1, 16)]
            digit = sum((c < need).astype(jnp.int32) for c in cnts)
            jstar = jstar | (digit << sh)

        sel = above | (tie & (idx <= jstar))
        norm = jnp.sqrt(jnp.sum(wp * wp)) + 1e-16
        wgt = jnp.where(sel, jnp.tanh(zd / norm), 0.0)

        acc = jnp.zeros((1, C), dtype=jnp.float32)
        for r in range(R):
            ptr = pts_ref[r]
            predt_ref[:, pl.ds(r * M, M)] = ptr
            acc = acc + jax.lax.dot_general(
                wgt[r:r + 1, :], ptr, (((1,), (1,)), ((), ())),
                preferred_element_type=jnp.float32)
        xf_ref[:, :] = acc * (1.0 / K)


@functools.partial(jax.jit, static_argnames=())
def kernel(x, edge_index, batch, W, b, w_pool):
    del edge_index, batch
    bt = b.reshape(C, 1)
    wp2 = w_pool.reshape(1, C)
    x_final, predt = pl.pallas_call(
        _body,
        grid=(R,),
        in_specs=[
            pl.BlockSpec((M, D), lambda j: (j, 0)),
            pl.BlockSpec((C, D), lambda j: (0, 0)),
            pl.BlockSpec((C, 1), lambda j: (0, 0)),
            pl.BlockSpec((1, C), lambda j: (0, 0)),
        ],
        out_specs=(
            pl.BlockSpec((1, C), lambda j: (0, 0)),
            pl.BlockSpec((C, N), lambda j: (0, 0)),
        ),
        out_shape=(
            jax.ShapeDtypeStruct((1, C), jnp.float32),
            jax.ShapeDtypeStruct((C, N), jnp.float32),
        ),
        scratch_shapes=[
            pltpu.VMEM((R, C, M), jnp.float32),
            pltpu.VMEM((R, M), jnp.float32),
        ],
    )(x, W, bt, wp2)
    return (x_final, predt.T)


# speedup vs baseline: 1.0038x; 1.0038x over previous
import functools

import jax
import jax.numpy as jnp
from jax.experimental import pallas as pl
from jax.experimental.pallas import tpu as pltpu

N = 10000
D = 128
C = 16
K = 5000
R = 10
M = N // R


def _in_cp(x_hbm, xb_ref, sems, j):
    return pltpu.make_async_copy(
        x_hbm.at[pl.ds(j * M, M), :], xb_ref.at[j % 2], sems.at[j % 2])


def _body(x_hbm, w_ref, b_ref, wp_ref, xf_ref, predt_hbm,
          xb_ref, pts_ref, ks_ref, ss_ref, sems, osem):
    w = w_ref[:, :]
    bt = b_ref[:, :]
    wp = wp_ref[:, :]

    _in_cp(x_hbm, xb_ref, sems, 0).start()

    c1 = [jnp.int32(0)] * 15

    for j in range(R):
        if j + 1 < R:
            _in_cp(x_hbm, xb_ref, sems, j + 1).start()
        _in_cp(x_hbm, xb_ref, sems, j).wait()

        xb = xb_ref[j % 2]
        pt = jax.lax.dot_general(
            w, xb, (((1,), (1,)), ((), ())),
            preferred_element_type=jnp.float32) + bt
        pts_ref[:, pl.ds(j * M, M)] = pt

        zj = jax.lax.dot_general(
            wp, pt, (((1,), (0,)), ((), ())),
            preferred_element_type=jnp.float32)
        u = jax.lax.bitcast_convert_type(zj, jnp.uint32)
        sign = u >> jnp.uint32(31)
        flip = jnp.where(sign == jnp.uint32(1),
                         jnp.uint32(0xFFFFFFFF), jnp.uint32(0x80000000))
        kj = u ^ flip
        ks_ref[pl.ds(j, 1), :] = kj
        norm = jnp.sqrt(jnp.sum(wp * wp)) + 1e-16
        ss_ref[pl.ds(j, 1), :] = jnp.tanh(zj / norm)
        c1 = [c + jnp.sum((kj >= jnp.uint32(d << 28)).astype(jnp.int32))
              for d, c in zip(range(1, 16), c1)]

    out_cp = pltpu.make_async_copy(pts_ref, predt_hbm, osem)
    out_cp.start()

    key = ks_ref[:, :]

    def _cnt_ge(t):
        return jnp.sum((key >= t).astype(jnp.int32))

    kth = sum((c >= K).astype(jnp.uint32) for c in c1) << jnp.uint32(28)
    for sh in range(24, -1, -4):
        cnts = [_cnt_ge(kth | jnp.uint32(d << sh)) for d in range(1, 16)]
        digit = sum((c >= K).astype(jnp.uint32) for c in cnts)
        kth = kth | (digit << jnp.uint32(sh))

    above = key > kth
    m = jnp.sum(above.astype(jnp.int32))
    need = K - m

    tie = key == kth
    idx = (jax.lax.broadcasted_iota(jnp.int32, (R, M), 0) * M
           + jax.lax.broadcasted_iota(jnp.int32, (R, M), 1))

    def _cnt_lt(v):
        return jnp.sum((tie & (idx < v)).astype(jnp.int32))

    jstar = jnp.int32(0)
    for sh in range(12, -1, -4):
        cnts = [_cnt_lt(jstar | jnp.int32(d << sh)) for d in range(1, 16)]
        digit = sum((c < need).astype(jnp.int32) for c in cnts)
        jstar = jstar | (digit << sh)

    sel = above | (tie & (idx <= jstar))
    wgt = jnp.where(sel, ss_ref[:, :], 0.0)

    acc = jnp.zeros((1, C), dtype=jnp.float32)
    for r in range(R):
        acc = acc + jax.lax.dot_general(
            wgt[r:r + 1, :], pts_ref[:, pl.ds(r * M, M)], (((1,), (1,)), ((), ())),
            preferred_element_type=jnp.float32)
    xf_ref[:, :] = acc * (1.0 / K)

    out_cp.wait()


@functools.partial(jax.jit, static_argnames=())
def kernel(x, edge_index, batch, W, b, w_pool):
    del edge_index, batch
    bt = b.reshape(C, 1)
    wp2 = w_pool.reshape(1, C)
    x_final, predt = pl.pallas_call(
        _body,
        in_specs=[
            pl.BlockSpec(memory_space=pl.ANY),
            pl.BlockSpec(memory_space=pltpu.MemorySpace.VMEM),
            pl.BlockSpec(memory_space=pltpu.MemorySpace.VMEM),
            pl.BlockSpec(memory_space=pltpu.MemorySpace.VMEM),
        ],
        out_specs=(
            pl.BlockSpec(memory_space=pltpu.MemorySpace.VMEM),
            pl.BlockSpec(memory_space=pl.ANY),
        ),
        out_shape=(
            jax.ShapeDtypeStruct((1, C), jnp.float32),
            jax.ShapeDtypeStruct((C, N), jnp.float32),
        ),
        scratch_shapes=[
            pltpu.VMEM((2, M, D), jnp.float32),
            pltpu.VMEM((C, N), jnp.float32),
            pltpu.VMEM((R, M), jnp.uint32),
            pltpu.VMEM((R, M), jnp.float32),
            pltpu.SemaphoreType.DMA((2,)),
            pltpu.SemaphoreType.DMA,
        ],
    )(x, W, bt, wp2)
    return (x_final, predt.T)
